# baseline (speedup 1.0000x reference)
import jax
import jax.numpy as jnp
from jax.experimental import pallas as pl
from jax.experimental.pallas import tpu as pltpu

_B, _C, _H, _W = 8, 384, 200, 176
_O_PAD = 32
_HB = 8


def _head_kernel(x_ref, w_ref, b_ref, cls_ref, reg_ref, dir_ref):
    v = x_ref[0].reshape(_C * _HB, _W)
    acc = jax.lax.dot_general(
        w_ref[...], v,
        dimension_numbers=(((1,), (0,)), ((), ())),
        preferred_element_type=jnp.float32,
    )
    acc = (acc + b_ref[...]).reshape(_HB, _O_PAD, _W)
    for h in range(_HB):
        a = acc[h]
        cls_ref[0, :, h, :] = a[0:2]
        reg_ref[0, :, h, :] = a[2:16]
        dir_ref[0, :, h, :] = a[16:20]


def kernel(x, W_cls, b_cls, W_reg, b_reg, W_dir, b_dir):
    w = jnp.concatenate([W_cls, W_reg, W_dir], axis=1).T
    w = jnp.pad(w, ((0, _O_PAD - w.shape[0]), (0, 0)))
    b = jnp.concatenate([b_cls, b_reg, b_dir])
    b = jnp.pad(b, (0, _O_PAD - b.shape[0]))

    eye = jnp.eye(_HB, dtype=jnp.float32)
    w_exp = (eye[:, None, None, :] * w[None, :, :, None]).reshape(
        _HB * _O_PAD, _C * _HB
    )
    b_exp = jnp.tile(b, _HB)[:, None]

    n_blocks = _H // _HB

    cls_o, reg_o, dir_o = pl.pallas_call(
        _head_kernel,
        grid=(_B, n_blocks),
        in_specs=[
            pl.BlockSpec((1, _C, _HB, _W), lambda bi, hi: (bi, 0, hi, 0)),
            pl.BlockSpec((_HB * _O_PAD, _C * _HB), lambda bi, hi: (0, 0)),
            pl.BlockSpec((_HB * _O_PAD, 1), lambda bi, hi: (0, 0)),
        ],
        out_specs=[
            pl.BlockSpec((1, 2, _HB, _W), lambda bi, hi: (bi, 0, hi, 0)),
            pl.BlockSpec((1, 14, _HB, _W), lambda bi, hi: (bi, 0, hi, 0)),
            pl.BlockSpec((1, 4, _HB, _W), lambda bi, hi: (bi, 0, hi, 0)),
        ],
        out_shape=[
            jax.ShapeDtypeStruct((_B, 2, _H, _W), jnp.float32),
            jax.ShapeDtypeStruct((_B, 14, _H, _W), jnp.float32),
            jax.ShapeDtypeStruct((_B, 4, _H, _W), jnp.float32),
        ],
        compiler_params=pltpu.CompilerParams(
            dimension_semantics=("parallel", "parallel"),
        ),
    )(x, w_exp, b_exp)

    return (cls_o, reg_o, dir_o)


# speedup vs baseline: 1.0268x; 1.0268x over previous
import jax
import jax.numpy as jnp
from jax.experimental import pallas as pl
from jax.experimental.pallas import tpu as pltpu

_B, _C, _H, _W = 8, 384, 200, 176
_O_PAD = 32
_CB = 64
_CI = _C // _CB
_G = _H // 8


def _head_kernel(x_ref, w_ref, b_ref, cls_ref, reg_ref, dir_ref, acc_s):
    ci = pl.program_id(1)

    @pl.when(ci == 0)
    def _():
        acc_s[...] = jnp.zeros_like(acc_s)

    wblk = w_ref[...]
    for g in range(_G):
        vg = x_ref[0, :, g * 8:(g + 1) * 8, :].reshape(_CB * 8, _W)
        acc_s[g] += jax.lax.dot_general(
            wblk, vg,
            dimension_numbers=(((1,), (0,)), ((), ())),
            preferred_element_type=jnp.float32,
        )

    @pl.when(ci == _CI - 1)
    def _():
        b = b_ref[...]
        for g in range(_G):
            a = acc_s[g] + b
            cls_ref[0, :, g * 8:(g + 1) * 8, :] = a[0:16].reshape(2, 8, _W)
            reg_ref[0, :, g * 8:(g + 1) * 8, :] = a[16:128].reshape(14, 8, _W)
            dir_ref[0, :, g * 8:(g + 1) * 8, :] = a[128:160].reshape(4, 8, _W)


def kernel(x, W_cls, b_cls, W_reg, b_reg, W_dir, b_dir):
    w = jnp.concatenate([W_cls, W_reg, W_dir], axis=1).T
    w = jnp.pad(w, ((0, _O_PAD - w.shape[0]), (0, 0)))
    b = jnp.concatenate([b_cls, b_reg, b_dir])
    b = jnp.pad(b, (0, _O_PAD - b.shape[0]))

    eye = jnp.eye(8, dtype=jnp.float32)
    w_exp = (w[:, None, :, None] * eye[None, :, None, :]).reshape(
        _O_PAD * 8, _C * 8
    )
    b_exp = jnp.repeat(b, 8)[:, None]

    cls_o, reg_o, dir_o = pl.pallas_call(
        _head_kernel,
        grid=(_B, _CI),
        in_specs=[
            pl.BlockSpec((1, _CB, _H, _W), lambda bi, ci: (bi, ci, 0, 0)),
            pl.BlockSpec((_O_PAD * 8, _CB * 8), lambda bi, ci: (0, ci)),
            pl.BlockSpec((_O_PAD * 8, 1), lambda bi, ci: (0, 0)),
        ],
        out_specs=[
            pl.BlockSpec((1, 2, _H, _W), lambda bi, ci: (bi, 0, 0, 0)),
            pl.BlockSpec((1, 14, _H, _W), lambda bi, ci: (bi, 0, 0, 0)),
            pl.BlockSpec((1, 4, _H, _W), lambda bi, ci: (bi, 0, 0, 0)),
        ],
        out_shape=[
            jax.ShapeDtypeStruct((_B, 2, _H, _W), jnp.float32),
            jax.ShapeDtypeStruct((_B, 14, _H, _W), jnp.float32),
            jax.ShapeDtypeStruct((_B, 4, _H, _W), jnp.float32),
        ],
        scratch_shapes=[
            pltpu.VMEM((_G, _O_PAD * 8, _W), jnp.float32),
        ],
        compiler_params=pltpu.CompilerParams(
            dimension_semantics=("parallel", "arbitrary"),
        ),
    )(x, w_exp, b_exp)

    return (cls_o, reg_o, dir_o)


# speedup vs baseline: 5.1367x; 5.0024x over previous
import jax
import jax.numpy as jnp
from jax.experimental import pallas as pl
from jax.experimental.pallas import tpu as pltpu

_B, _C, _H, _W = 8, 384, 200, 176
_O_PAD = 32
_HB = 40


def _head_kernel(x_ref, w_ref, b_ref, cls_ref, reg_ref, dir_ref):
    xm = x_ref[0].reshape(_HB * _W, _C)
    acc = jax.lax.dot_general(
        xm.astype(jnp.bfloat16), w_ref[...],
        dimension_numbers=(((1,), (0,)), ((), ())),
        preferred_element_type=jnp.float32,
    ) + b_ref[...]
    for h in range(_HB):
        t = jnp.transpose(acc[h * _W:(h + 1) * _W])
        cls_ref[0, :, h, :] = t[0:2]
        reg_ref[0, :, h, :] = t[2:16]
        dir_ref[0, :, h, :] = t[16:20]


def kernel(x, W_cls, b_cls, W_reg, b_reg, W_dir, b_dir):
    w = jnp.concatenate([W_cls, W_reg, W_dir], axis=1)
    w = jnp.pad(w, ((0, 0), (0, _O_PAD - w.shape[1])))
    b = jnp.concatenate([b_cls, b_reg, b_dir])
    b = jnp.pad(b, (0, _O_PAD - b.shape[0]))[None, :]

    xt = jnp.transpose(x, (0, 2, 3, 1))
    n_blocks = _H // _HB

    cls_o, reg_o, dir_o = pl.pallas_call(
        _head_kernel,
        grid=(_B, n_blocks),
        in_specs=[
            pl.BlockSpec((1, _HB, _W, _C), lambda bi, hi: (bi, hi, 0, 0)),
            pl.BlockSpec((_C, _O_PAD), lambda bi, hi: (0, 0)),
            pl.BlockSpec((1, _O_PAD), lambda bi, hi: (0, 0)),
        ],
        out_specs=[
            pl.BlockSpec((1, 2, _HB, _W), lambda bi, hi: (bi, 0, hi, 0)),
            pl.BlockSpec((1, 14, _HB, _W), lambda bi, hi: (bi, 0, hi, 0)),
            pl.BlockSpec((1, 4, _HB, _W), lambda bi, hi: (bi, 0, hi, 0)),
        ],
        out_shape=[
            jax.ShapeDtypeStruct((_B, 2, _H, _W), jnp.float32),
            jax.ShapeDtypeStruct((_B, 14, _H, _W), jnp.float32),
            jax.ShapeDtypeStruct((_B, 4, _H, _W), jnp.float32),
        ],
        compiler_params=pltpu.CompilerParams(
            dimension_semantics=("parallel", "parallel"),
        ),
    )(xt, w.astype(jnp.bfloat16), b)

    return (cls_o, reg_o, dir_o)


# speedup vs baseline: 5.1746x; 1.0074x over previous
import jax
import jax.numpy as jnp
from jax.experimental import pallas as pl
from jax.experimental.pallas import tpu as pltpu

_B, _C, _H, _W = 8, 384, 200, 176
_O_PAD = 32
_HB = 40


def _head_kernel(x_ref, w_ref, b_ref, cls_ref, reg_ref, dir_ref):
    xm = x_ref[0].reshape(_HB * _W, _C)
    acc = jax.lax.dot_general(
        xm, w_ref[...],
        dimension_numbers=(((1,), (0,)), ((), ())),
        preferred_element_type=jnp.float32,
        precision=jax.lax.Precision.DEFAULT,
    ) + b_ref[...]
    for h in range(_HB):
        t = jnp.transpose(acc[h * _W:(h + 1) * _W])
        cls_ref[0, :, h, :] = t[0:2]
        reg_ref[0, :, h, :] = t[2:16]
        dir_ref[0, :, h, :] = t[16:20]


def kernel(x, W_cls, b_cls, W_reg, b_reg, W_dir, b_dir):
    w = jnp.concatenate([W_cls, W_reg, W_dir], axis=1)
    w = jnp.pad(w, ((0, 0), (0, _O_PAD - w.shape[1])))
    b = jnp.concatenate([b_cls, b_reg, b_dir])
    b = jnp.pad(b, (0, _O_PAD - b.shape[0]))[None, :]

    xt = jnp.transpose(x, (0, 2, 3, 1))
    n_blocks = _H // _HB

    cls_o, reg_o, dir_o = pl.pallas_call(
        _head_kernel,
        grid=(_B, n_blocks),
        in_specs=[
            pl.BlockSpec((1, _HB, _W, _C), lambda bi, hi: (bi, hi, 0, 0)),
            pl.BlockSpec((_C, _O_PAD), lambda bi, hi: (0, 0)),
            pl.BlockSpec((1, _O_PAD), lambda bi, hi: (0, 0)),
        ],
        out_specs=[
            pl.BlockSpec((1, 2, _HB, _W), lambda bi, hi: (bi, 0, hi, 0)),
            pl.BlockSpec((1, 14, _HB, _W), lambda bi, hi: (bi, 0, hi, 0)),
            pl.BlockSpec((1, 4, _HB, _W), lambda bi, hi: (bi, 0, hi, 0)),
        ],
        out_shape=[
            jax.ShapeDtypeStruct((_B, 2, _H, _W), jnp.float32),
            jax.ShapeDtypeStruct((_B, 14, _H, _W), jnp.float32),
            jax.ShapeDtypeStruct((_B, 4, _H, _W), jnp.float32),
        ],
        compiler_params=pltpu.CompilerParams(
            dimension_semantics=("parallel", "parallel"),
        ),
    )(xt, w, b)

    return (cls_o, reg_o, dir_o)
